# Initial kernel scaffold; baseline (speedup 1.0000x reference)
import jax
import jax.numpy as jnp
from jax.experimental import pallas as pl


def kernel(x, edge_index, aux_features, conv1_w, conv1_b, conv2_w, conv2_b, fcfe_w, fcfe_b, gcn1_w, gcn1_b, gcn2_w, gcn2_b, fc_w, fc_b, out_w, out_b):
    raise NotImplementedError("write your pallas kernel here")


# speedup vs baseline: 1.9791x; 1.9791x over previous
import functools

import jax
import jax.numpy as jnp
from jax import lax
from jax.experimental import pallas as pl
from jax.experimental.pallas import tpu as pltpu
from jax.experimental.pallas import tpu_sc as plsc

_NC = 2
_NS = 16
_K = 128
_R = 2000

_HI = lax.Precision.HIGHEST


def _conv_nb(x, w, pad):
    return lax.conv_general_dilated(
        x, w, (1, 1), [(pad, pad), (pad, pad)],
        dimension_numbers=('NCHW', 'OIHW', 'NCHW'))


def _pool_axis(t, axis):
    size = t.shape[axis]
    segs = []
    for i in range(2):
        s = (i * size) // 2
        e = -(-((i + 1) * size) // 2)
        sl = lax.slice_in_dim(t, s, e, axis=axis)
        segs.append(sl.mean(axis=axis, keepdims=True))
    return jnp.concatenate(segs, axis=axis)


def _fold_weights(conv1_w, conv1_b, conv2_w, conv2_b, fcfe_w, fcfe_b, gcn1_w):
    eye9 = jnp.eye(9, dtype=jnp.float32).reshape(9, 1, 3, 3)
    w1 = _conv_nb(eye9, conv1_w, 1).reshape(9, 288)
    b1 = jnp.repeat(conv1_b, 9)
    eye288 = jnp.eye(288, dtype=jnp.float32).reshape(288, 32, 3, 3)
    w2 = _conv_nb(eye288, conv2_w, 1).reshape(288, 1024)
    b2 = jnp.repeat(conv2_b, 16)
    eye1024 = jnp.eye(1024, dtype=jnp.float32).reshape(1024, 64, 4, 4)
    pool = _pool_axis(_pool_axis(eye1024, 2), 3).reshape(1024, 256)
    wf = pool @ (fcfe_w @ gcn1_w)
    bf = fcfe_b @ gcn1_w
    w1p = jnp.zeros((16, 288), jnp.float32).at[:9].set(w1)
    return w1p, b1, w2, b2, wf, bf


def _acc_geom(n):
    opt = -(-(-(-n // _NS)) // 8) * 8
    n_out = _NS * opt
    zpt = -(-max(n_out, n + 1) // _NS // 8) * 8
    n_acc = _NS * zpt
    return opt, n_out, zpt, n_acc


def _sc_s1(scat_idx, ones_rows, zrows, n, e_pad):
    opt, n_out, zpt, n_acc = _acc_geom(n)
    chunks = e_pad // (_NS * _K)
    mesh = plsc.VectorSubcoreMesh(core_axis_name="c", subcore_axis_name="s",
                                  num_cores=_NC, num_subcores=_NS)

    @functools.partial(
        pl.kernel, mesh=mesh,
        out_type=[jax.ShapeDtypeStruct((n_out, 16), jnp.float32),
                  jax.ShapeDtypeStruct((n_out, 16), jnp.float32)],
        scratch_types=[
            pltpu.VMEM_SHARED((n_acc, 16), jnp.float32),
            pltpu.VMEM((_K,), jnp.int32),
            pltpu.VMEM((_K, 16), jnp.float32),
        ],
        compiler_params=pltpu.CompilerParams(use_tc_tiling_on_sc=False))
    def s1(scat_idx_hbm, ones_hbm, zrows_hbm, deg_hbm, cnt_hbm,
           acc, idx_v, ones_v):
        cid = lax.axis_index("c")
        sid = lax.axis_index("s")
        pltpu.sync_copy(zrows_hbm, acc.at[pl.ds(sid * zpt, zpt)])
        pltpu.sync_copy(ones_hbm, ones_v)
        plsc.subcore_barrier()
        for j in range(chunks):
            off = cid * e_pad + sid * (chunks * _K) + j * _K
            pltpu.sync_copy(scat_idx_hbm.at[pl.ds(off, _K)], idx_v)
            pltpu.sync_copy(ones_v, acc.at[idx_v], add=True)
        plsc.subcore_barrier()

        @pl.when(cid == 0)
        def _():
            pltpu.sync_copy(acc.at[pl.ds(sid * opt, opt)],
                            deg_hbm.at[pl.ds(sid * opt, opt)])

        @pl.when(cid == 1)
        def _():
            pltpu.sync_copy(acc.at[pl.ds(sid * opt, opt)],
                            cnt_hbm.at[pl.ds(sid * opt, opt)])

    return s1(scat_idx, ones_rows, zrows)


def _sc_agg(ya, yb, src_g, dst_g, dst_s, nodeinfo, zrows, n, e_pad, with_info):
    opt, n_out, zpt, n_acc = _acc_geom(n)
    chunks = e_pad // (_NS * _K)
    mesh = plsc.VectorSubcoreMesh(core_axis_name="c", subcore_axis_name="s",
                                  num_cores=_NC, num_subcores=_NS)
    out_type = [jax.ShapeDtypeStruct((n_out, 32), jnp.float32),
                jax.ShapeDtypeStruct((n_out, 32), jnp.float32)]
    if with_info:
        out_type += [jax.ShapeDtypeStruct((e_pad, 16), jnp.float32),
                     jax.ShapeDtypeStruct((e_pad, 16), jnp.float32)]

    @functools.partial(
        pl.kernel, mesh=mesh,
        out_type=out_type,
        scratch_types=[
            pltpu.VMEM_SHARED((n_acc, 32), jnp.float32),
            pltpu.VMEM((_K,), jnp.int32),
            pltpu.VMEM((_K,), jnp.int32),
            pltpu.VMEM((_K,), jnp.int32),
            pltpu.VMEM((_K, 32), jnp.float32),
            pltpu.VMEM((_K, 16), jnp.float32),
            pltpu.SemaphoreType.DMA,
        ],
        compiler_params=pltpu.CompilerParams(use_tc_tiling_on_sc=False))
    def agg_kernel(ya_hbm, yb_hbm, src_g_hbm, dst_g_hbm, dst_s_hbm,
                   info_hbm, zrows_hbm, *out_and_scratch):
        if with_info:
            agga_hbm, aggb_hbm, isrc_hbm, idst_hbm = out_and_scratch[:4]
            rest = out_and_scratch[4:]
        else:
            agga_hbm, aggb_hbm = out_and_scratch[:2]
            rest = out_and_scratch[2:]
        acc, sidx_v, didx_v, gidx_v, rows_v, info_v, sem = rest
        cid = lax.axis_index("c")
        sid = lax.axis_index("s")
        pltpu.sync_copy(zrows_hbm, acc.at[pl.ds(sid * zpt, zpt)])
        plsc.subcore_barrier()
        for j in range(chunks):
            off = sid * (chunks * _K) + j * _K
            pltpu.sync_copy(src_g_hbm.at[pl.ds(off, _K)], sidx_v)
            pltpu.sync_copy(dst_s_hbm.at[pl.ds(off, _K)], didx_v)

            @pl.when(cid == 0)
            def _():
                pltpu.async_copy(ya_hbm.at[sidx_v], rows_v, sem).wait()

            @pl.when(cid == 1)
            def _():
                pltpu.async_copy(yb_hbm.at[sidx_v], rows_v, sem).wait()

            pltpu.sync_copy(rows_v, acc.at[didx_v], add=True)
            if with_info:
                @pl.when(cid == 0)
                def _():
                    pltpu.async_copy(info_hbm.at[sidx_v], info_v, sem).wait()
                    pltpu.sync_copy(info_v, isrc_hbm.at[pl.ds(off, _K)])

                @pl.when(cid == 1)
                def _():
                    pltpu.sync_copy(dst_g_hbm.at[pl.ds(off, _K)], gidx_v)
                    pltpu.async_copy(info_hbm.at[gidx_v], info_v, sem).wait()
                    pltpu.sync_copy(info_v, idst_hbm.at[pl.ds(off, _K)])
        plsc.subcore_barrier()

        @pl.when(cid == 0)
        def _():
            pltpu.sync_copy(acc.at[pl.ds(sid * opt, opt)],
                            agga_hbm.at[pl.ds(sid * opt, opt)])

        @pl.when(cid == 1)
        def _():
            pltpu.sync_copy(acc.at[pl.ds(sid * opt, opt)],
                            aggb_hbm.at[pl.ds(sid * opt, opt)])

    return agg_kernel(ya, yb, src_g, dst_g, dst_s, nodeinfo, zrows)


def _tc_mlp(x16, w1p, b1, w2, b2, wf, bf, n):
    nb = n // _R

    def body(x_ref, w1_ref, b1_ref, w2_ref, b2_ref, wf_ref, bf_ref, o_ref):
        h1 = jnp.maximum(
            jnp.dot(x_ref[...], w1_ref[...], precision=_HI) + b1_ref[...], 0.0)
        h2 = jnp.maximum(
            jnp.dot(h1, w2_ref[...], precision=_HI) + b2_ref[...], 0.0)
        o_ref[...] = jnp.dot(h2, wf_ref[...], precision=_HI) + bf_ref[...]

    return pl.pallas_call(
        body,
        grid=(nb,),
        in_specs=[
            pl.BlockSpec((_R, 16), lambda i: (i, 0)),
            pl.BlockSpec((16, 288), lambda i: (0, 0)),
            pl.BlockSpec((1, 288), lambda i: (0, 0)),
            pl.BlockSpec((288, 1024), lambda i: (0, 0)),
            pl.BlockSpec((1, 1024), lambda i: (0, 0)),
            pl.BlockSpec((1024, 64), lambda i: (0, 0)),
            pl.BlockSpec((1, 64), lambda i: (0, 0)),
        ],
        out_specs=pl.BlockSpec((_R, 64), lambda i: (i, 0)),
        out_shape=jax.ShapeDtypeStruct((n, 64), jnp.float32),
    )(x16, w1p, b1.reshape(1, 288), w2, b2.reshape(1, 1024), wf,
      bf.reshape(1, 64))


def _tc_prep(aux, deg16, cnt16, xw1, n):
    nb = n // _R

    def body(aux_ref, deg_ref, cnt_ref, xw_ref, info_ref, ya_ref, yb_ref):
        deg = deg_ref[:, 0:1] + 1.0
        cnt = cnt_ref[:, 0:1]
        dinv = lax.rsqrt(deg)
        theta = aux_ref[:, 0:1]
        r = aux_ref[:, 1:2]
        frame = aux_ref[:, 2:3]
        px = r * jnp.cos(theta)
        py = r * jnp.sin(theta)
        zero = jnp.zeros((_R, 11), jnp.float32)
        info_ref[...] = jnp.concatenate([frame, cnt, px, py, dinv, zero],
                                        axis=1)
        y1 = dinv * xw_ref[...]
        ya_ref[...] = y1[:, :32]
        yb_ref[...] = y1[:, 32:]

    return pl.pallas_call(
        body,
        grid=(nb,),
        in_specs=[
            pl.BlockSpec((_R, 3), lambda i: (i, 0)),
            pl.BlockSpec((_R, 16), lambda i: (i, 0)),
            pl.BlockSpec((_R, 16), lambda i: (i, 0)),
            pl.BlockSpec((_R, 64), lambda i: (i, 0)),
        ],
        out_specs=[
            pl.BlockSpec((_R, 16), lambda i: (i, 0)),
            pl.BlockSpec((_R, 32), lambda i: (i, 0)),
            pl.BlockSpec((_R, 32), lambda i: (i, 0)),
        ],
        out_shape=[
            jax.ShapeDtypeStruct((n, 16), jnp.float32),
            jax.ShapeDtypeStruct((n, 32), jnp.float32),
            jax.ShapeDtypeStruct((n, 32), jnp.float32),
        ],
    )(aux, deg16, cnt16, xw1)


def _tc_mid(agg1a, agg1b, ya, yb, nodeinfo, isrc, idst, gcn1_b, gcn2_w, n):
    nb = n // _R

    def body(aggA_ref, aggB_ref, ya_ref, yb_ref, info_ref, is_ref, id_ref,
             b1_ref, w2_ref, y2a_ref, y2b_ref, ex_ref):
        dinv = info_ref[:, 4:5]
        agg = jnp.concatenate([aggA_ref[...], aggB_ref[...]], axis=1)
        y1 = jnp.concatenate([ya_ref[...], yb_ref[...]], axis=1)
        g1 = dinv * (agg + y1) + b1_ref[...]
        fs = is_ref[:, 0:1]
        cs = is_ref[:, 1:2]
        pxs = is_ref[:, 2:3]
        pys = is_ref[:, 3:4]
        fd = id_ref[:, 0:1]
        pxd = id_ref[:, 2:3]
        pyd = id_ref[:, 3:4]
        df = jnp.abs(fs - fd)
        a1 = jnp.where(df == 1.0, 1.0 / jnp.maximum(cs, 1.0), 0.0)
        x1 = jnp.maximum(g1 * a1, 0.0)
        d2 = (pxd - pxs) ** 2 + (pyd - pys) ** 2
        disp = jnp.sqrt(jnp.maximum(d2, 1e-12))
        vel = disp / jnp.where(df == 2.0, df, 1.0)
        a2 = jnp.where(df == 2.0, jnp.exp(-vel / 8.5), 0.0)
        y2 = dinv * jnp.dot(x1, w2_ref[...], precision=_HI)
        y2a_ref[...] = y2[:, :32]
        y2b_ref[...] = y2[:, 32:]
        zero = jnp.zeros((_R, 6), jnp.float32)
        ex_ref[...] = jnp.concatenate([a2, dinv, zero], axis=1)

    return pl.pallas_call(
        body,
        grid=(nb,),
        in_specs=[
            pl.BlockSpec((_R, 32), lambda i: (i, 0)),
            pl.BlockSpec((_R, 32), lambda i: (i, 0)),
            pl.BlockSpec((_R, 32), lambda i: (i, 0)),
            pl.BlockSpec((_R, 32), lambda i: (i, 0)),
            pl.BlockSpec((_R, 16), lambda i: (i, 0)),
            pl.BlockSpec((_R, 16), lambda i: (i, 0)),
            pl.BlockSpec((_R, 16), lambda i: (i, 0)),
            pl.BlockSpec((1, 64), lambda i: (0, 0)),
            pl.BlockSpec((64, 64), lambda i: (0, 0)),
        ],
        out_specs=[
            pl.BlockSpec((_R, 32), lambda i: (i, 0)),
            pl.BlockSpec((_R, 32), lambda i: (i, 0)),
            pl.BlockSpec((_R, 8), lambda i: (i, 0)),
        ],
        out_shape=[
            jax.ShapeDtypeStruct((n, 32), jnp.float32),
            jax.ShapeDtypeStruct((n, 32), jnp.float32),
            jax.ShapeDtypeStruct((n, 8), jnp.float32),
        ],
    )(agg1a, agg1b, ya, yb, nodeinfo, isrc, idst,
      gcn1_b.reshape(1, 64), gcn2_w)


def _tc_head(agg2a, agg2b, y2a, y2b, extras, gcn2_b, fc_w, fc_b,
             out_w, out_b, n):
    nb = n // _R

    def body(aggA_ref, aggB_ref, ya_ref, yb_ref, ex_ref, b2_ref, fw_ref,
             fb_ref, ow_ref, ob_ref, o_ref):
        a2 = ex_ref[:, 0:1]
        dinv = ex_ref[:, 1:2]
        agg = jnp.concatenate([aggA_ref[...], aggB_ref[...]], axis=1)
        y2 = jnp.concatenate([ya_ref[...], yb_ref[...]], axis=1)
        g2 = dinv * (agg + y2) + b2_ref[...]
        x2 = jnp.maximum(g2 * a2, 0.0)
        h = jnp.maximum(jnp.dot(x2, fw_ref[...], precision=_HI) + fb_ref[...],
                        0.0)
        logit = jnp.dot(h, ow_ref[...], precision=_HI) + ob_ref[...]
        o_ref[...] = 1.0 / (1.0 + jnp.exp(-logit))

    return pl.pallas_call(
        body,
        grid=(nb,),
        in_specs=[
            pl.BlockSpec((_R, 32), lambda i: (i, 0)),
            pl.BlockSpec((_R, 32), lambda i: (i, 0)),
            pl.BlockSpec((_R, 32), lambda i: (i, 0)),
            pl.BlockSpec((_R, 32), lambda i: (i, 0)),
            pl.BlockSpec((_R, 8), lambda i: (i, 0)),
            pl.BlockSpec((1, 64), lambda i: (0, 0)),
            pl.BlockSpec((64, 32), lambda i: (0, 0)),
            pl.BlockSpec((1, 32), lambda i: (0, 0)),
            pl.BlockSpec((32, 1), lambda i: (0, 0)),
            pl.BlockSpec((1, 1), lambda i: (0, 0)),
        ],
        out_specs=pl.BlockSpec((_R, 1), lambda i: (i, 0)),
        out_shape=jax.ShapeDtypeStruct((n, 1), jnp.float32),
    )(agg2a, agg2b, y2a, y2b, extras, gcn2_b.reshape(1, 64), fc_w,
      fc_b.reshape(1, 32), out_w, out_b.reshape(1, 1))


def kernel(x, edge_index, aux_features, conv1_w, conv1_b, conv2_w, conv2_b,
           fcfe_w, fcfe_b, gcn1_w, gcn1_b, gcn2_w, gcn2_b,
           fc_w, fc_b, out_w, out_b):
    n = x.shape[0]
    e = edge_index.shape[1]
    e_pad = -(-e // (_NS * _K)) * (_NS * _K)

    x16 = jnp.zeros((n, 16), jnp.float32).at[:, :9].set(x.reshape(n, 9))
    w1p, b1, w2, b2, wf, bf = _fold_weights(
        conv1_w, conv1_b, conv2_w, conv2_b, fcfe_w, fcfe_b, gcn1_w)
    src = edge_index[0]
    dst = edge_index[1]
    pad = e_pad - e
    pad_g = jnp.zeros((pad,), jnp.int32)
    pad_s = jnp.full((pad,), n, jnp.int32)
    src_g = jnp.concatenate([src, pad_g])
    dst_g = jnp.concatenate([dst, pad_g])
    src_s = jnp.concatenate([src, pad_s])
    dst_s = jnp.concatenate([dst, pad_s])
    scat_idx = jnp.concatenate([dst_s, src_s])
    ones_rows = jnp.zeros((_K, 16), jnp.float32).at[:, 0].set(1.0)
    zpt = _acc_geom(n)[2]
    zrows16 = jnp.zeros((zpt, 16), jnp.float32)
    zrows32 = jnp.zeros((zpt, 32), jnp.float32)

    deg16, cnt16 = _sc_s1(scat_idx, ones_rows, zrows16, n, e_pad)
    xw1 = _tc_mlp(x16, w1p, b1, w2, b2, wf, bf, n)
    nodeinfo, y1a, y1b = _tc_prep(aux_features, deg16, cnt16, xw1, n)
    agg1a, agg1b, isrc, idst = _sc_agg(y1a, y1b, src_g, dst_g, dst_s,
                                       nodeinfo, zrows32, n, e_pad,
                                       with_info=True)
    y2a, y2b, extras = _tc_mid(agg1a, agg1b, y1a, y1b, nodeinfo, isrc, idst,
                               gcn1_b, gcn2_w, n)
    agg2a, agg2b = _sc_agg(y2a, y2b, src_g, dst_g, dst_s, nodeinfo,
                           zrows32, n, e_pad, with_info=False)
    out = _tc_head(agg2a, agg2b, y2a, y2b, extras, gcn2_b, fc_w, fc_b,
                   out_w, out_b, n)
    return out


# speedup vs baseline: 3.0962x; 1.5644x over previous
import functools

import jax
import jax.numpy as jnp
from jax import lax
from jax.experimental import pallas as pl
from jax.experimental.pallas import tpu as pltpu
from jax.experimental.pallas import tpu_sc as plsc

_NC = 2
_NS = 16
_K = 128
_R = 2000

_HI = lax.Precision.HIGHEST


def _conv_nb(x, w, pad):
    return lax.conv_general_dilated(
        x, w, (1, 1), [(pad, pad), (pad, pad)],
        dimension_numbers=('NCHW', 'OIHW', 'NCHW'))


def _pool_axis(t, axis):
    size = t.shape[axis]
    segs = []
    for i in range(2):
        s = (i * size) // 2
        e = -(-((i + 1) * size) // 2)
        sl = lax.slice_in_dim(t, s, e, axis=axis)
        segs.append(sl.mean(axis=axis, keepdims=True))
    return jnp.concatenate(segs, axis=axis)


def _fold_weights(conv1_w, conv1_b, conv2_w, conv2_b, fcfe_w, fcfe_b, gcn1_w):
    eye9 = jnp.eye(9, dtype=jnp.float32).reshape(9, 1, 3, 3)
    w1 = _conv_nb(eye9, conv1_w, 1).reshape(9, 288)
    b1 = jnp.repeat(conv1_b, 9)
    eye288 = jnp.eye(288, dtype=jnp.float32).reshape(288, 32, 3, 3)
    w2 = _conv_nb(eye288, conv2_w, 1).reshape(288, 1024)
    b2 = jnp.repeat(conv2_b, 16)
    eye1024 = jnp.eye(1024, dtype=jnp.float32).reshape(1024, 64, 4, 4)
    pool = _pool_axis(_pool_axis(eye1024, 2), 3).reshape(1024, 256)
    wf = pool @ (fcfe_w @ gcn1_w)
    bf = fcfe_b @ gcn1_w
    w1p = jnp.zeros((16, 288), jnp.float32).at[:9].set(w1)
    return w1p, b1, w2, b2, wf, bf


def _acc_geom(n):
    opt = -(-(-(-n // _NS)) // 8) * 8
    n_out = _NS * opt
    zpt = -(-max(n_out, n + 1) // _NS // 8) * 8
    n_acc = _NS * zpt
    return opt, n_out, zpt, n_acc


def _sc_s1(scat_idx, ones_rows, zrows, n, e_pad):
    opt, n_out, zpt, n_acc = _acc_geom(n)
    chunks = e_pad // (_NS * _K)
    mesh = plsc.VectorSubcoreMesh(core_axis_name="c", subcore_axis_name="s",
                                  num_cores=_NC, num_subcores=_NS)

    @functools.partial(
        pl.kernel, mesh=mesh,
        out_type=[jax.ShapeDtypeStruct((n_out, 16), jnp.float32),
                  jax.ShapeDtypeStruct((n_out, 16), jnp.float32)],
        scratch_types=[
            pltpu.VMEM_SHARED((n_acc, 16), jnp.float32),
            pltpu.VMEM((_K,), jnp.int32),
            pltpu.VMEM((_K, 16), jnp.float32),
        ],
        compiler_params=pltpu.CompilerParams(use_tc_tiling_on_sc=False))
    def s1(scat_idx_hbm, ones_hbm, zrows_hbm, deg_hbm, cnt_hbm,
           acc, idx_v, ones_v):
        cid = lax.axis_index("c")
        sid = lax.axis_index("s")
        pltpu.sync_copy(zrows_hbm, acc.at[pl.ds(sid * zpt, zpt)])
        pltpu.sync_copy(ones_hbm, ones_v)
        plsc.subcore_barrier()
        for j in range(chunks):
            off = cid * e_pad + sid * (chunks * _K) + j * _K
            pltpu.sync_copy(scat_idx_hbm.at[pl.ds(off, _K)], idx_v)
            pltpu.sync_copy(ones_v, acc.at[idx_v], add=True)
        plsc.subcore_barrier()

        @pl.when(cid == 0)
        def _():
            pltpu.sync_copy(acc.at[pl.ds(sid * opt, opt)],
                            deg_hbm.at[pl.ds(sid * opt, opt)])

        @pl.when(cid == 1)
        def _():
            pltpu.sync_copy(acc.at[pl.ds(sid * opt, opt)],
                            cnt_hbm.at[pl.ds(sid * opt, opt)])

    return s1(scat_idx, ones_rows, zrows)


def _sc_agg(ya, yb, src_g, dst_g, dst_s, nodeinfo, zrows, n, e_pad, with_info):
    opt, n_out, zpt, n_acc = _acc_geom(n)
    chunks = e_pad // (_NS * _K)
    mesh = plsc.VectorSubcoreMesh(core_axis_name="c", subcore_axis_name="s",
                                  num_cores=_NC, num_subcores=_NS)
    out_type = [jax.ShapeDtypeStruct((n_out, 32), jnp.float32),
                jax.ShapeDtypeStruct((n_out, 32), jnp.float32)]
    if with_info:
        out_type += [jax.ShapeDtypeStruct((e_pad, 16), jnp.float32),
                     jax.ShapeDtypeStruct((e_pad, 16), jnp.float32)]

    @functools.partial(
        pl.kernel, mesh=mesh,
        out_type=out_type,
        scratch_types=[
            pltpu.VMEM_SHARED((n_acc, 32), jnp.float32),
            pltpu.VMEM((_K,), jnp.int32),
            pltpu.VMEM((_K,), jnp.int32),
            pltpu.VMEM((_K,), jnp.int32),
            pltpu.VMEM((_K, 32), jnp.float32),
            pltpu.VMEM((_K, 16), jnp.float32),
            pltpu.SemaphoreType.DMA,
        ],
        compiler_params=pltpu.CompilerParams(use_tc_tiling_on_sc=False))
    def agg_kernel(ya_hbm, yb_hbm, src_g_hbm, dst_g_hbm, dst_s_hbm,
                   info_hbm, zrows_hbm, *out_and_scratch):
        if with_info:
            agga_hbm, aggb_hbm, isrc_hbm, idst_hbm = out_and_scratch[:4]
            rest = out_and_scratch[4:]
        else:
            agga_hbm, aggb_hbm = out_and_scratch[:2]
            rest = out_and_scratch[2:]
        acc, sidx_v, didx_v, gidx_v, rows_v, info_v, sem = rest
        cid = lax.axis_index("c")
        sid = lax.axis_index("s")
        pltpu.sync_copy(zrows_hbm, acc.at[pl.ds(sid * zpt, zpt)])
        plsc.subcore_barrier()
        for j in range(chunks):
            off = sid * (chunks * _K) + j * _K
            pltpu.sync_copy(src_g_hbm.at[pl.ds(off, _K)], sidx_v)
            pltpu.sync_copy(dst_s_hbm.at[pl.ds(off, _K)], didx_v)

            @pl.when(cid == 0)
            def _():
                pltpu.async_copy(ya_hbm.at[sidx_v], rows_v, sem).wait()

            @pl.when(cid == 1)
            def _():
                pltpu.async_copy(yb_hbm.at[sidx_v], rows_v, sem).wait()

            pltpu.sync_copy(rows_v, acc.at[didx_v], add=True)
            if with_info:
                @pl.when(cid == 0)
                def _():
                    pltpu.async_copy(info_hbm.at[sidx_v], info_v, sem).wait()
                    pltpu.sync_copy(info_v, isrc_hbm.at[pl.ds(off, _K)])

                @pl.when(cid == 1)
                def _():
                    pltpu.sync_copy(dst_g_hbm.at[pl.ds(off, _K)], gidx_v)
                    pltpu.async_copy(info_hbm.at[gidx_v], info_v, sem).wait()
                    pltpu.sync_copy(info_v, idst_hbm.at[pl.ds(off, _K)])
        plsc.subcore_barrier()

        @pl.when(cid == 0)
        def _():
            pltpu.sync_copy(acc.at[pl.ds(sid * opt, opt)],
                            agga_hbm.at[pl.ds(sid * opt, opt)])

        @pl.when(cid == 1)
        def _():
            pltpu.sync_copy(acc.at[pl.ds(sid * opt, opt)],
                            aggb_hbm.at[pl.ds(sid * opt, opt)])

    return agg_kernel(ya, yb, src_g, dst_g, dst_s, nodeinfo, zrows)


def _tc_mlp(x16, w1p, b1, w2, b2, wf, bf, n):
    nb = n // _R

    def body(x_ref, w1_ref, b1_ref, w2_ref, b2_ref, wf_ref, bf_ref, o_ref):
        h1 = jnp.maximum(
            jnp.dot(x_ref[...], w1_ref[...],
                    preferred_element_type=jnp.float32) + b1_ref[...], 0.0)
        h2 = jnp.maximum(
            jnp.dot(h1.astype(jnp.bfloat16), w2_ref[...],
                    preferred_element_type=jnp.float32) + b2_ref[...], 0.0)
        o_ref[...] = jnp.dot(h2.astype(jnp.bfloat16), wf_ref[...],
                             preferred_element_type=jnp.float32) + bf_ref[...]

    return pl.pallas_call(
        body,
        grid=(nb,),
        in_specs=[
            pl.BlockSpec((_R, 16), lambda i: (i, 0)),
            pl.BlockSpec((16, 288), lambda i: (0, 0)),
            pl.BlockSpec((1, 288), lambda i: (0, 0)),
            pl.BlockSpec((288, 1024), lambda i: (0, 0)),
            pl.BlockSpec((1, 1024), lambda i: (0, 0)),
            pl.BlockSpec((1024, 64), lambda i: (0, 0)),
            pl.BlockSpec((1, 64), lambda i: (0, 0)),
        ],
        out_specs=pl.BlockSpec((_R, 64), lambda i: (i, 0)),
        out_shape=jax.ShapeDtypeStruct((n, 64), jnp.float32),
    )(x16, w1p, b1.reshape(1, 288), w2.astype(jnp.bfloat16),
      b2.reshape(1, 1024), wf.astype(jnp.bfloat16), bf.reshape(1, 64))


def _tc_prep(aux, deg16, cnt16, xw1, n):
    nb = n // _R

    def body(aux_ref, deg_ref, cnt_ref, xw_ref, info_ref, ya_ref, yb_ref):
        deg = deg_ref[:, 0:1] + 1.0
        cnt = cnt_ref[:, 0:1]
        dinv = lax.rsqrt(deg)
        theta = aux_ref[:, 0:1]
        r = aux_ref[:, 1:2]
        frame = aux_ref[:, 2:3]
        px = r * jnp.cos(theta)
        py = r * jnp.sin(theta)
        zero = jnp.zeros((_R, 11), jnp.float32)
        info_ref[...] = jnp.concatenate([frame, cnt, px, py, dinv, zero],
                                        axis=1)
        y1 = dinv * xw_ref[...]
        ya_ref[...] = y1[:, :32]
        yb_ref[...] = y1[:, 32:]

    return pl.pallas_call(
        body,
        grid=(nb,),
        in_specs=[
            pl.BlockSpec((_R, 3), lambda i: (i, 0)),
            pl.BlockSpec((_R, 16), lambda i: (i, 0)),
            pl.BlockSpec((_R, 16), lambda i: (i, 0)),
            pl.BlockSpec((_R, 64), lambda i: (i, 0)),
        ],
        out_specs=[
            pl.BlockSpec((_R, 16), lambda i: (i, 0)),
            pl.BlockSpec((_R, 32), lambda i: (i, 0)),
            pl.BlockSpec((_R, 32), lambda i: (i, 0)),
        ],
        out_shape=[
            jax.ShapeDtypeStruct((n, 16), jnp.float32),
            jax.ShapeDtypeStruct((n, 32), jnp.float32),
            jax.ShapeDtypeStruct((n, 32), jnp.float32),
        ],
    )(aux, deg16, cnt16, xw1)


def _tc_mid(agg1a, agg1b, ya, yb, nodeinfo, isrc, idst, gcn1_b, gcn2_w, n):
    nb = n // _R

    def body(aggA_ref, aggB_ref, ya_ref, yb_ref, info_ref, is_ref, id_ref,
             b1_ref, w2_ref, y2a_ref, y2b_ref, ex_ref):
        dinv = info_ref[:, 4:5]
        agg = jnp.concatenate([aggA_ref[...], aggB_ref[...]], axis=1)
        y1 = jnp.concatenate([ya_ref[...], yb_ref[...]], axis=1)
        g1 = dinv * (agg + y1) + b1_ref[...]
        fs = is_ref[:, 0:1]
        cs = is_ref[:, 1:2]
        pxs = is_ref[:, 2:3]
        pys = is_ref[:, 3:4]
        fd = id_ref[:, 0:1]
        pxd = id_ref[:, 2:3]
        pyd = id_ref[:, 3:4]
        df = jnp.abs(fs - fd)
        a1 = jnp.where(df == 1.0, 1.0 / jnp.maximum(cs, 1.0), 0.0)
        x1 = jnp.maximum(g1 * a1, 0.0)
        d2 = (pxd - pxs) ** 2 + (pyd - pys) ** 2
        disp = jnp.sqrt(jnp.maximum(d2, 1e-12))
        vel = disp / jnp.where(df == 2.0, df, 1.0)
        a2 = jnp.where(df == 2.0, jnp.exp(-vel / 8.5), 0.0)
        y2 = dinv * jnp.dot(x1, w2_ref[...], precision=_HI)
        y2a_ref[...] = y2[:, :32]
        y2b_ref[...] = y2[:, 32:]
        zero = jnp.zeros((_R, 6), jnp.float32)
        ex_ref[...] = jnp.concatenate([a2, dinv, zero], axis=1)

    return pl.pallas_call(
        body,
        grid=(nb,),
        in_specs=[
            pl.BlockSpec((_R, 32), lambda i: (i, 0)),
            pl.BlockSpec((_R, 32), lambda i: (i, 0)),
            pl.BlockSpec((_R, 32), lambda i: (i, 0)),
            pl.BlockSpec((_R, 32), lambda i: (i, 0)),
            pl.BlockSpec((_R, 16), lambda i: (i, 0)),
            pl.BlockSpec((_R, 16), lambda i: (i, 0)),
            pl.BlockSpec((_R, 16), lambda i: (i, 0)),
            pl.BlockSpec((1, 64), lambda i: (0, 0)),
            pl.BlockSpec((64, 64), lambda i: (0, 0)),
        ],
        out_specs=[
            pl.BlockSpec((_R, 32), lambda i: (i, 0)),
            pl.BlockSpec((_R, 32), lambda i: (i, 0)),
            pl.BlockSpec((_R, 8), lambda i: (i, 0)),
        ],
        out_shape=[
            jax.ShapeDtypeStruct((n, 32), jnp.float32),
            jax.ShapeDtypeStruct((n, 32), jnp.float32),
            jax.ShapeDtypeStruct((n, 8), jnp.float32),
        ],
    )(agg1a, agg1b, ya, yb, nodeinfo, isrc, idst,
      gcn1_b.reshape(1, 64), gcn2_w)


def _tc_head(agg2a, agg2b, y2a, y2b, extras, gcn2_b, fc_w, fc_b,
             out_w, out_b, n):
    nb = n // _R

    def body(aggA_ref, aggB_ref, ya_ref, yb_ref, ex_ref, b2_ref, fw_ref,
             fb_ref, ow_ref, ob_ref, o_ref):
        a2 = ex_ref[:, 0:1]
        dinv = ex_ref[:, 1:2]
        agg = jnp.concatenate([aggA_ref[...], aggB_ref[...]], axis=1)
        y2 = jnp.concatenate([ya_ref[...], yb_ref[...]], axis=1)
        g2 = dinv * (agg + y2) + b2_ref[...]
        x2 = jnp.maximum(g2 * a2, 0.0)
        h = jnp.maximum(jnp.dot(x2, fw_ref[...], precision=_HI) + fb_ref[...],
                        0.0)
        logit = jnp.dot(h, ow_ref[...], precision=_HI) + ob_ref[...]
        o_ref[...] = 1.0 / (1.0 + jnp.exp(-logit))

    return pl.pallas_call(
        body,
        grid=(nb,),
        in_specs=[
            pl.BlockSpec((_R, 32), lambda i: (i, 0)),
            pl.BlockSpec((_R, 32), lambda i: (i, 0)),
            pl.BlockSpec((_R, 32), lambda i: (i, 0)),
            pl.BlockSpec((_R, 32), lambda i: (i, 0)),
            pl.BlockSpec((_R, 8), lambda i: (i, 0)),
            pl.BlockSpec((1, 64), lambda i: (0, 0)),
            pl.BlockSpec((64, 32), lambda i: (0, 0)),
            pl.BlockSpec((1, 32), lambda i: (0, 0)),
            pl.BlockSpec((32, 1), lambda i: (0, 0)),
            pl.BlockSpec((1, 1), lambda i: (0, 0)),
        ],
        out_specs=pl.BlockSpec((_R, 1), lambda i: (i, 0)),
        out_shape=jax.ShapeDtypeStruct((n, 1), jnp.float32),
    )(agg2a, agg2b, y2a, y2b, extras, gcn2_b.reshape(1, 64), fc_w,
      fc_b.reshape(1, 32), out_w, out_b.reshape(1, 1))


def kernel(x, edge_index, aux_features, conv1_w, conv1_b, conv2_w, conv2_b,
           fcfe_w, fcfe_b, gcn1_w, gcn1_b, gcn2_w, gcn2_b,
           fc_w, fc_b, out_w, out_b):
    n = x.shape[0]
    e = edge_index.shape[1]
    e_pad = -(-e // (_NS * _K)) * (_NS * _K)

    x16 = jnp.zeros((n, 16), jnp.float32).at[:, :9].set(x.reshape(n, 9))
    w1p, b1, w2, b2, wf, bf = _fold_weights(
        conv1_w, conv1_b, conv2_w, conv2_b, fcfe_w, fcfe_b, gcn1_w)
    src = edge_index[0]
    dst = edge_index[1]
    pad = e_pad - e
    pad_g = jnp.zeros((pad,), jnp.int32)
    pad_s = jnp.full((pad,), n, jnp.int32)
    src_g = jnp.concatenate([src, pad_g])
    dst_g = jnp.concatenate([dst, pad_g])
    src_s = jnp.concatenate([src, pad_s])
    dst_s = jnp.concatenate([dst, pad_s])
    scat_idx = jnp.concatenate([dst_s, src_s])
    ones_rows = jnp.zeros((_K, 16), jnp.float32).at[:, 0].set(1.0)
    zpt = _acc_geom(n)[2]
    zrows16 = jnp.zeros((zpt, 16), jnp.float32)
    zrows32 = jnp.zeros((zpt, 32), jnp.float32)

    deg16, cnt16 = _sc_s1(scat_idx, ones_rows, zrows16, n, e_pad)
    xw1 = _tc_mlp(x16, w1p, b1, w2, b2, wf, bf, n)
    nodeinfo, y1a, y1b = _tc_prep(aux_features, deg16, cnt16, xw1, n)
    agg1a, agg1b, isrc, idst = _sc_agg(y1a, y1b, src_g, dst_g, dst_s,
                                       nodeinfo, zrows32, n, e_pad,
                                       with_info=True)
    y2a, y2b, extras = _tc_mid(agg1a, agg1b, y1a, y1b, nodeinfo, isrc, idst,
                               gcn1_b, gcn2_w, n)
    agg2a, agg2b = _sc_agg(y2a, y2b, src_g, dst_g, dst_s, nodeinfo,
                           zrows32, n, e_pad, with_info=False)
    out = _tc_head(agg2a, agg2b, y2a, y2b, extras, gcn2_b, fc_w, fc_b,
                   out_w, out_b, n)
    return out
